# Initial kernel scaffold; baseline (speedup 1.0000x reference)
import jax
import jax.numpy as jnp
from jax.experimental import pallas as pl


def kernel(pred, target, prev_target, k, x, dt, edge_index):
    raise NotImplementedError("write your pallas kernel here")


# speedup vs baseline: 27.3649x; 27.3649x over previous
import functools

import jax
import jax.numpy as jnp
from jax import lax
from jax.experimental import pallas as pl
from jax.experimental.pallas import tpu as pltpu
from jax.experimental.pallas import tpu_sc as plsc

NC = 2
NS = 16
NW = NC * NS
GROUP = 128
BLK = 512


def _tc_tables_body(c_ref, p_ref, t_ref, v_ref, ud_ref, du_ref, dsum_ref):
    i = pl.program_id(0)
    c0 = c_ref[0]
    c1 = c_ref[1]
    c2 = c_ref[2]
    p = p_ref[...]
    t = t_ref[...]
    v = v_ref[...]
    diff = p - t
    part = jnp.sum(diff * diff)

    @pl.when(i == 0)
    def _():
        dsum_ref[0, 0] = 0.0

    dsum_ref[0, 0] += part
    u = c0 * p + c1 * v
    d = p - c2 * v
    ud_ref[...] = jnp.concatenate([u, d], axis=0)
    du_ref[...] = jnp.concatenate([d, u], axis=0)


def _make_tc_tables(b2, npad):
    grid = npad // BLK
    return pl.pallas_call(
        _tc_tables_body,
        grid=(grid,),
        in_specs=[
            pl.BlockSpec(memory_space=pltpu.SMEM),
            pl.BlockSpec((b2 // 2, BLK), lambda i: (0, i)),
            pl.BlockSpec((b2 // 2, BLK), lambda i: (0, i)),
            pl.BlockSpec((b2 // 2, BLK), lambda i: (0, i)),
        ],
        out_specs=[
            pl.BlockSpec((b2, BLK), lambda i: (0, i)),
            pl.BlockSpec((b2, BLK), lambda i: (0, i)),
            pl.BlockSpec((1, 1), lambda i: (0, 0), memory_space=pltpu.SMEM),
        ],
        out_shape=[
            jax.ShapeDtypeStruct((b2, npad), jnp.float32),
            jax.ShapeDtypeStruct((b2, npad), jnp.float32),
            jax.ShapeDtypeStruct((1, 1), jnp.float32),
        ],
    )


def _make_sc_phy(epw):
    groups = epw // GROUP
    mesh = plsc.VectorSubcoreMesh(core_axis_name="c", subcore_axis_name="s")

    @functools.partial(
        pl.kernel,
        mesh=mesh,
        compiler_params=pltpu.CompilerParams(use_tc_tiling_on_sc=False),
        out_type=jax.ShapeDtypeStruct((NW, 16), jnp.float32),
        scratch_types=[
            pltpu.VMEM((GROUP,), jnp.int32),
            pltpu.VMEM((GROUP,), jnp.int32),
            pltpu.VMEM((GROUP, 16), jnp.float32),
            pltpu.VMEM((GROUP, 16), jnp.float32),
            pltpu.VMEM((16,), jnp.float32),
            pltpu.SemaphoreType.DMA,
            pltpu.SemaphoreType.DMA,
        ],
    )
    def sc_phy(t_hbm, t2_hbm, src_hbm, dst_hbm, out_hbm,
               idx_s, idx_d, rows_s, rows_d, accv, sem1, sem2):
        wid = lax.axis_index("s") * NC + lax.axis_index("c")
        base = wid * epw

        def group_body(g, acc):
            gb = base + g * GROUP
            pltpu.sync_copy(src_hbm.at[pl.ds(gb, GROUP)], idx_s)
            pltpu.sync_copy(dst_hbm.at[pl.ds(gb, GROUP)], idx_d)
            cp1 = pltpu.async_copy(t_hbm.at[idx_s], rows_s, sem1)
            cp2 = pltpu.async_copy(t2_hbm.at[idx_d], rows_d, sem2)
            cp1.wait()
            cp2.wait()
            gsum = jnp.zeros((16,), jnp.float32)
            for i in range(GROUP):
                r = rows_d[i, :] - rows_s[i, :]
                gsum = gsum + r * r
            return acc + gsum

        acc = lax.fori_loop(0, groups, group_body, jnp.zeros((16,), jnp.float32))
        accv[...] = acc
        pltpu.sync_copy(accv, out_hbm.at[wid])

    return sc_phy


def kernel(pred, target, prev_target, k, x, dt, edge_index):
    b, n = pred.shape[0], pred.shape[1]
    e = edge_index.shape[1]
    b2 = 2 * b

    denom = 2.0 * k * (1.0 - x) + dt
    c0 = (dt - 2.0 * k * x) / denom
    c1 = (dt + 2.0 * k * x) / denom
    c2 = (2.0 * k * (1.0 - x) - dt) / denom
    cvec = jnp.stack([c0, c1, c2]).astype(jnp.float32)

    npad = ((n + 1 + BLK - 1) // BLK) * BLK
    pred2 = pred[:, :, 0]
    padn = ((0, 0), (0, npad - n))
    pred_p = jnp.pad(pred2, padn)
    targ_p = jnp.pad(target[:, :, 0], padn)
    prev_p = jnp.pad(prev_target, padn)

    ud, du, dsum = _make_tc_tables(b2, npad)(cvec, pred_p, targ_p, prev_p)
    t_tab = ud.T
    t2_tab = du.T

    epw = ((e + NW * GROUP - 1) // (NW * GROUP)) * GROUP
    epad = NW * epw
    pad_e = epad - e
    src_p = jnp.concatenate(
        [edge_index[0], jnp.full((pad_e,), n, dtype=jnp.int32)])
    dst_p = jnp.concatenate(
        [edge_index[1], jnp.full((pad_e,), n, dtype=jnp.int32)])

    acc = _make_sc_phy(epw)(t_tab, t2_tab, src_p, dst_p)

    lane_sums = jnp.sum(acc, axis=0)
    per_batch = lane_sums[:b] / jnp.float32(e)
    phy_loss = jnp.mean(per_batch)
    data_loss = dsum[0, 0] / jnp.float32(b * n)
    total = data_loss + phy_loss
    return (total, data_loss, phy_loss)


# speedup vs baseline: 73.9059x; 2.7008x over previous
import functools

import jax
import jax.numpy as jnp
from jax import lax
from jax.experimental import pallas as pl
from jax.experimental.pallas import tpu as pltpu
from jax.experimental.pallas import tpu_sc as plsc

NC = 2
NS = 16
NW = NC * NS
GROUP = 128
GG = 8
SCE = GROUP * GG
BLK = 512
UNROLL = 16


def _tc_tables_body(c_ref, p_ref, t_ref, v_ref, ud_ref, du_ref, dsum_ref):
    i = pl.program_id(0)
    c0 = c_ref[0]
    c1 = c_ref[1]
    c2 = c_ref[2]
    p = p_ref[...]
    t = t_ref[...]
    v = v_ref[...]
    diff = p - t
    part = jnp.sum(diff * diff)

    @pl.when(i == 0)
    def _():
        dsum_ref[0, 0] = 0.0

    dsum_ref[0, 0] += part
    u = c0 * p + c1 * v
    d = p - c2 * v
    ud_ref[...] = jnp.concatenate([u, d], axis=0)
    du_ref[...] = jnp.concatenate([d, u], axis=0)


def _make_tc_tables(b2, npad):
    grid = npad // BLK
    return pl.pallas_call(
        _tc_tables_body,
        grid=(grid,),
        in_specs=[
            pl.BlockSpec(memory_space=pltpu.SMEM),
            pl.BlockSpec((b2 // 2, BLK), lambda i: (0, i)),
            pl.BlockSpec((b2 // 2, BLK), lambda i: (0, i)),
            pl.BlockSpec((b2 // 2, BLK), lambda i: (0, i)),
        ],
        out_specs=[
            pl.BlockSpec((b2, BLK), lambda i: (0, i)),
            pl.BlockSpec((b2, BLK), lambda i: (0, i)),
            pl.BlockSpec((1, 1), lambda i: (0, 0), memory_space=pltpu.SMEM),
        ],
        out_shape=[
            jax.ShapeDtypeStruct((b2, npad), jnp.float32),
            jax.ShapeDtypeStruct((b2, npad), jnp.float32),
            jax.ShapeDtypeStruct((1, 1), jnp.float32),
        ],
    )


def _make_sc_phy(epw):
    nsc = epw // SCE
    assert epw % SCE == 0 and nsc % 2 == 0
    mesh = plsc.VectorSubcoreMesh(core_axis_name="c", subcore_axis_name="s")

    @functools.partial(
        pl.kernel,
        mesh=mesh,
        compiler_params=pltpu.CompilerParams(use_tc_tiling_on_sc=False),
        out_type=jax.ShapeDtypeStruct((NW, 16), jnp.float32),
        scratch_types=[
            pltpu.VMEM((2, SCE), jnp.int32),
            pltpu.VMEM((2, SCE), jnp.int32),
            pltpu.VMEM((2 * GG, GROUP, 16), jnp.float32),
            pltpu.VMEM((2 * GG, GROUP, 16), jnp.float32),
            pltpu.VMEM((16,), jnp.float32),
            pltpu.SemaphoreType.DMA,
            pltpu.SemaphoreType.DMA,
        ],
    )
    def sc_phy(t_hbm, t2_hbm, src_hbm, dst_hbm, out_hbm,
               idx_s, idx_d, rows_s, rows_d, accv, sem_a, sem_b):
        wid = lax.axis_index("s") * NC + lax.axis_index("c")
        base = wid * epw
        sems = (sem_a, sem_b)

        def gathers(bufb):
            cps = []
            for j in range(GG):
                isl = idx_s.at[bufb, pl.ds(j * GROUP, GROUP)]
                dsl = idx_d.at[bufb, pl.ds(j * GROUP, GROUP)]
                cps.append(pltpu.make_async_copy(
                    t_hbm.at[isl], rows_s.at[bufb * GG + j], sems[bufb]))
                cps.append(pltpu.make_async_copy(
                    t2_hbm.at[dsl], rows_d.at[bufb * GG + j], sems[bufb]))
            return cps

        def prefetch(c, bufb):
            gb = base + c * SCE
            pltpu.sync_copy(src_hbm.at[pl.ds(gb, SCE)], idx_s.at[bufb])
            pltpu.sync_copy(dst_hbm.at[pl.ds(gb, SCE)], idx_d.at[bufb])
            for cp in gathers(bufb):
                cp.start()

        def compute(bufb, acc):
            gsum = jnp.zeros((16,), jnp.float32)
            for j in range(GG):
                slot = bufb * GG + j

                def ibody(iv, g, slot=slot):
                    for u_ in range(UNROLL):
                        i = iv * UNROLL + u_
                        r = rows_d[slot, i, :] - rows_s[slot, i, :]
                        g = g + r * r
                    return g

                gsum = lax.fori_loop(0, GROUP // UNROLL, ibody, gsum)
            return acc + gsum

        prefetch(0, 0)
        prefetch(1, 1)

        def pair_body(cc, acc):
            c = 2 * cc
            for bufb in range(2):
                for cp in gathers(bufb):
                    cp.wait()
                acc = compute(bufb, acc)

                @pl.when(c + 2 + bufb < nsc)
                def _(c=c, bufb=bufb):
                    prefetch(c + 2 + bufb, bufb)
            return acc

        acc = lax.fori_loop(0, nsc // 2, pair_body,
                            jnp.zeros((16,), jnp.float32))
        accv[...] = acc
        pltpu.sync_copy(accv, out_hbm.at[wid])

    return sc_phy


def kernel(pred, target, prev_target, k, x, dt, edge_index):
    b, n = pred.shape[0], pred.shape[1]
    e = edge_index.shape[1]
    b2 = 2 * b

    denom = 2.0 * k * (1.0 - x) + dt
    c0 = (dt - 2.0 * k * x) / denom
    c1 = (dt + 2.0 * k * x) / denom
    c2 = (2.0 * k * (1.0 - x) - dt) / denom
    cvec = jnp.stack([c0, c1, c2]).astype(jnp.float32)

    npad = ((n + 1 + BLK - 1) // BLK) * BLK
    padn = ((0, 0), (0, npad - n))
    pred_p = jnp.pad(pred[:, :, 0], padn)
    targ_p = jnp.pad(target[:, :, 0], padn)
    prev_p = jnp.pad(prev_target, padn)

    ud, du, dsum = _make_tc_tables(b2, npad)(cvec, pred_p, targ_p, prev_p)
    t_tab = ud.T
    t2_tab = du.T

    epw = ((e + NW * 2 * SCE - 1) // (NW * 2 * SCE)) * 2 * SCE
    epad = NW * epw
    pad_e = epad - e
    src_p = jnp.concatenate(
        [edge_index[0], jnp.full((pad_e,), n, dtype=jnp.int32)])
    dst_p = jnp.concatenate(
        [edge_index[1], jnp.full((pad_e,), n, dtype=jnp.int32)])

    acc = _make_sc_phy(epw)(t_tab, t2_tab, src_p, dst_p)

    lane_sums = jnp.sum(acc, axis=0)
    per_batch = lane_sums[:b] / jnp.float32(e)
    phy_loss = jnp.mean(per_batch)
    data_loss = dsum[0, 0] / jnp.float32(b * n)
    total = data_loss + phy_loss
    return (total, data_loss, phy_loss)


# speedup vs baseline: 88.4318x; 1.1965x over previous
import functools

import jax
import jax.numpy as jnp
from jax import lax
from jax.experimental import pallas as pl
from jax.experimental.pallas import tpu as pltpu
from jax.experimental.pallas import tpu_sc as plsc

NC = 2
NS = 16
NW = NC * NS
GROUP = 128
GG = 10
SCE = GROUP * GG
BLK = 512
UNROLL = 16


def _tc_tables_body(n, c_ref, p_ref, t_ref, v_ref, tab_ref, tab2_ref, dsum_ref):
    i = pl.program_id(0)
    c0 = c_ref[0]
    c1 = c_ref[1]
    c2 = c_ref[2]
    p = p_ref[...]
    t = t_ref[...]
    v = v_ref[...]
    col = i * BLK + jax.lax.broadcasted_iota(jnp.int32, p.shape, 1)
    valid = col < n
    diff = jnp.where(valid, p - t, 0.0)
    part = jnp.sum(diff * diff)

    @pl.when(i == 0)
    def _():
        dsum_ref[0, 0] = 0.0

    dsum_ref[0, 0] += part
    u = c0 * p + c1 * v
    d = p - c2 * v
    ut = u.T
    dt_ = d.T
    tab_ref[...] = jnp.concatenate([ut, dt_], axis=1)
    tab2_ref[...] = jnp.concatenate([dt_, ut], axis=1)


def _make_tc_tables(b2, n):
    grid = (n + BLK - 1) // BLK
    return pl.pallas_call(
        functools.partial(_tc_tables_body, n),
        grid=(grid,),
        in_specs=[
            pl.BlockSpec(memory_space=pltpu.SMEM),
            pl.BlockSpec((b2 // 2, BLK), lambda i: (0, i)),
            pl.BlockSpec((b2 // 2, BLK), lambda i: (0, i)),
            pl.BlockSpec((b2 // 2, BLK), lambda i: (0, i)),
        ],
        out_specs=[
            pl.BlockSpec((BLK, b2), lambda i: (i, 0)),
            pl.BlockSpec((BLK, b2), lambda i: (i, 0)),
            pl.BlockSpec((1, 1), lambda i: (0, 0), memory_space=pltpu.SMEM),
        ],
        out_shape=[
            jax.ShapeDtypeStruct((n, b2), jnp.float32),
            jax.ShapeDtypeStruct((n, b2), jnp.float32),
            jax.ShapeDtypeStruct((1, 1), jnp.float32),
        ],
    )


def _make_sc_phy(epw):
    npair = epw // (2 * SCE)
    tail = epw - npair * 2 * SCE
    tail_groups = [GROUP] * (tail // GROUP)
    if tail % GROUP:
        tail_groups.append(tail % GROUP)
    assert len(tail_groups) <= 2 * GG
    mesh = plsc.VectorSubcoreMesh(core_axis_name="c", subcore_axis_name="s")

    @functools.partial(
        pl.kernel,
        mesh=mesh,
        compiler_params=pltpu.CompilerParams(use_tc_tiling_on_sc=False),
        out_type=jax.ShapeDtypeStruct((NW, 16), jnp.float32),
        scratch_types=[
            pltpu.VMEM((2, SCE), jnp.int32),
            pltpu.VMEM((2, SCE), jnp.int32),
            pltpu.VMEM((2 * GG, GROUP, 16), jnp.float32),
            pltpu.VMEM((2 * GG, GROUP, 16), jnp.float32),
            pltpu.VMEM((16,), jnp.float32),
            pltpu.SemaphoreType.DMA,
            pltpu.SemaphoreType.DMA,
        ],
    )
    def sc_phy(t_hbm, t2_hbm, src_hbm, dst_hbm, out_hbm,
               idx_s, idx_d, rows_s, rows_d, accv, sem_a, sem_b):
        wid = lax.axis_index("s") * NC + lax.axis_index("c")
        base = wid * epw
        sems = (sem_a, sem_b)

        def gathers(bufb):
            cps = []
            for j in range(GG):
                isl = idx_s.at[bufb, pl.ds(j * GROUP, GROUP)]
                dsl = idx_d.at[bufb, pl.ds(j * GROUP, GROUP)]
                cps.append(pltpu.make_async_copy(
                    t_hbm.at[isl], rows_s.at[bufb * GG + j], sems[bufb]))
                cps.append(pltpu.make_async_copy(
                    t2_hbm.at[dsl], rows_d.at[bufb * GG + j], sems[bufb]))
            return cps

        def prefetch(c, bufb):
            gb = base + c * SCE
            pltpu.sync_copy(src_hbm.at[pl.ds(gb, SCE)], idx_s.at[bufb])
            pltpu.sync_copy(dst_hbm.at[pl.ds(gb, SCE)], idx_d.at[bufb])
            for cp in gathers(bufb):
                cp.start()

        def compute_group(slot, m, gsum):
            def ibody(iv, g, slot=slot):
                for u_ in range(UNROLL):
                    i = iv * UNROLL + u_
                    r = rows_d[slot, i, :] - rows_s[slot, i, :]
                    g = g + r * r
                return g

            gsum = lax.fori_loop(0, m // UNROLL, ibody, gsum)
            for i in range(m - (m % UNROLL), m):
                r = rows_d[slot, i, :] - rows_s[slot, i, :]
                gsum = gsum + r * r
            return gsum

        def compute(bufb, acc):
            gsum = jnp.zeros((16,), jnp.float32)
            for j in range(GG):
                gsum = compute_group(bufb * GG + j, GROUP, gsum)
            return acc + gsum

        acc = jnp.zeros((16,), jnp.float32)
        if npair > 0:
            prefetch(0, 0)
            prefetch(1, 1)

            def pair_body(cc, acc):
                c = 2 * cc
                for bufb in range(2):
                    for cp in gathers(bufb):
                        cp.wait()
                    acc = compute(bufb, acc)

                    @pl.when(c + 2 + bufb < 2 * npair)
                    def _(c=c, bufb=bufb):
                        prefetch(c + 2 + bufb, bufb)
                return acc

            acc = lax.fori_loop(0, npair, pair_body, acc)

        if tail_groups:
            tb = base + npair * 2 * SCE
            tlen = sum(tail_groups)
            pltpu.sync_copy(src_hbm.at[pl.ds(tb, tlen)],
                            idx_s.at[0, pl.ds(0, tlen)])
            pltpu.sync_copy(dst_hbm.at[pl.ds(tb, tlen)],
                            idx_d.at[0, pl.ds(0, tlen)])
            cps = []
            off = 0
            for j, m in enumerate(tail_groups):
                isl = idx_s.at[0, pl.ds(off, m)]
                dsl = idx_d.at[0, pl.ds(off, m)]
                cps.append(pltpu.make_async_copy(
                    t_hbm.at[isl], rows_s.at[j, pl.ds(0, m)], sem_a))
                cps.append(pltpu.make_async_copy(
                    t2_hbm.at[dsl], rows_d.at[j, pl.ds(0, m)], sem_a))
                off += m
            for cp in cps:
                cp.start()
            for cp in cps:
                cp.wait()
            gsum = jnp.zeros((16,), jnp.float32)
            for j, m in enumerate(tail_groups):
                gsum = compute_group(j, m, gsum)
            acc = acc + gsum

        accv[...] = acc
        pltpu.sync_copy(accv, out_hbm.at[wid])

    return sc_phy


def kernel(pred, target, prev_target, k, x, dt, edge_index):
    b, n = pred.shape[0], pred.shape[1]
    e = edge_index.shape[1]
    b2 = 2 * b

    denom = 2.0 * k * (1.0 - x) + dt
    c0 = (dt - 2.0 * k * x) / denom
    c1 = (dt + 2.0 * k * x) / denom
    c2 = (2.0 * k * (1.0 - x) - dt) / denom
    cvec = jnp.stack([c0, c1, c2]).astype(jnp.float32)

    t_tab, t2_tab, dsum = _make_tc_tables(b2, n)(
        cvec, pred[:, :, 0], target[:, :, 0], prev_target)

    epad = ((e + NW * 8 - 1) // (NW * 8)) * (NW * 8)
    pad_e = epad - e
    src = edge_index[0]
    dst = edge_index[1]
    if pad_e:
        zpad = jnp.zeros((pad_e,), dtype=jnp.int32)
        src = jnp.concatenate([src, zpad])
        dst = jnp.concatenate([dst, zpad])

    acc = _make_sc_phy(epad // NW)(t_tab, t2_tab, src, dst)

    lane_sums = jnp.sum(acc, axis=0)
    per_batch = lane_sums[:b]
    if pad_e:
        r0 = t2_tab[0, :b] - t_tab[0, :b]
        per_batch = per_batch - jnp.float32(pad_e) * r0 * r0
    phy_loss = jnp.mean(per_batch / jnp.float32(e))
    data_loss = dsum[0, 0] / jnp.float32(b * n)
    total = data_loss + phy_loss
    return (total, data_loss, phy_loss)


# speedup vs baseline: 92.2361x; 1.0430x over previous
import functools

import jax
import jax.numpy as jnp
from jax import lax
from jax.experimental import pallas as pl
from jax.experimental.pallas import tpu as pltpu
from jax.experimental.pallas import tpu_sc as plsc

NC = 2
NS = 16
NW = NC * NS
GROUP = 128
GG = 10
SCE = GROUP * GG
BLK = 512
UNROLL = 16


def _tc_tables_body(n, c_ref, p_ref, t_ref, v_ref, tab_ref, tab2_ref, dsum_ref):
    i = pl.program_id(0)
    c0 = c_ref[0]
    c1 = c_ref[1]
    c2 = c_ref[2]
    p = p_ref[...]
    t = t_ref[...]
    v = v_ref[...]
    col = i * BLK + jax.lax.broadcasted_iota(jnp.int32, p.shape, 1)
    valid = col < n
    diff = jnp.where(valid, p - t, 0.0)
    part = jnp.sum(diff * diff)

    @pl.when(i == 0)
    def _():
        dsum_ref[0, 0] = 0.0

    dsum_ref[0, 0] += part
    u = c0 * p + c1 * v
    d = p - c2 * v
    ut = u.T
    dt_ = d.T
    tab_ref[...] = jnp.concatenate([ut, dt_], axis=1)
    tab2_ref[...] = jnp.concatenate([dt_, ut], axis=1)


def _make_tc_tables(b2, n):
    grid = (n + BLK - 1) // BLK
    rows = grid * BLK
    return pl.pallas_call(
        functools.partial(_tc_tables_body, n),
        grid=(grid,),
        in_specs=[
            pl.BlockSpec(memory_space=pltpu.SMEM),
            pl.BlockSpec((b2 // 2, BLK), lambda i: (0, i)),
            pl.BlockSpec((b2 // 2, BLK), lambda i: (0, i)),
            pl.BlockSpec((b2 // 2, BLK), lambda i: (0, i)),
        ],
        out_specs=[
            pl.BlockSpec((BLK, b2), lambda i: (i, 0)),
            pl.BlockSpec((BLK, b2), lambda i: (i, 0)),
            pl.BlockSpec((1, 1), lambda i: (0, 0), memory_space=pltpu.SMEM),
        ],
        out_shape=[
            jax.ShapeDtypeStruct((rows, b2), jnp.float32),
            jax.ShapeDtypeStruct((rows, b2), jnp.float32),
            jax.ShapeDtypeStruct((1, 1), jnp.float32),
        ],
    )


def _make_sc_phy(epw):
    npair = epw // (2 * SCE)
    tail = epw - npair * 2 * SCE
    tail_groups = [GROUP] * (tail // GROUP)
    if tail % GROUP:
        tail_groups.append(tail % GROUP)
    assert len(tail_groups) <= 2 * GG
    mesh = plsc.VectorSubcoreMesh(core_axis_name="c", subcore_axis_name="s")

    @functools.partial(
        pl.kernel,
        mesh=mesh,
        compiler_params=pltpu.CompilerParams(use_tc_tiling_on_sc=False),
        out_type=jax.ShapeDtypeStruct((NW, 16), jnp.float32),
        scratch_types=[
            pltpu.VMEM((2, SCE), jnp.int32),
            pltpu.VMEM((2, SCE), jnp.int32),
            pltpu.VMEM((2 * GG, GROUP, 16), jnp.float32),
            pltpu.VMEM((2 * GG, GROUP, 16), jnp.float32),
            pltpu.VMEM((16,), jnp.float32),
            pltpu.SemaphoreType.DMA,
            pltpu.SemaphoreType.DMA,
        ],
    )
    def sc_phy(t_hbm, t2_hbm, edge_hbm, out_hbm,
               idx_s, idx_d, rows_s, rows_d, accv, sem_a, sem_b):
        wid = lax.axis_index("s") * NC + lax.axis_index("c")
        base = wid * epw
        sems = (sem_a, sem_b)
        src_hbm = edge_hbm.at[0]
        dst_hbm = edge_hbm.at[1]

        def gathers(bufb):
            cps = []
            for j in range(GG):
                isl = idx_s.at[bufb, pl.ds(j * GROUP, GROUP)]
                dsl = idx_d.at[bufb, pl.ds(j * GROUP, GROUP)]
                cps.append(pltpu.make_async_copy(
                    t_hbm.at[isl], rows_s.at[bufb * GG + j], sems[bufb]))
                cps.append(pltpu.make_async_copy(
                    t2_hbm.at[dsl], rows_d.at[bufb * GG + j], sems[bufb]))
            return cps

        def prefetch(c, bufb):
            gb = base + c * SCE
            pltpu.sync_copy(src_hbm.at[pl.ds(gb, SCE)], idx_s.at[bufb])
            pltpu.sync_copy(dst_hbm.at[pl.ds(gb, SCE)], idx_d.at[bufb])
            for cp in gathers(bufb):
                cp.start()

        def compute_group(slot, m, gsum):
            def ibody(iv, g, slot=slot):
                for u_ in range(UNROLL):
                    i = iv * UNROLL + u_
                    r = rows_d[slot, i, :] - rows_s[slot, i, :]
                    g = g + r * r
                return g

            gsum = lax.fori_loop(0, m // UNROLL, ibody, gsum)
            for i in range(m - (m % UNROLL), m):
                r = rows_d[slot, i, :] - rows_s[slot, i, :]
                gsum = gsum + r * r
            return gsum

        def compute(bufb, acc):
            gsum = jnp.zeros((16,), jnp.float32)
            for j in range(GG):
                gsum = compute_group(bufb * GG + j, GROUP, gsum)
            return acc + gsum

        acc = jnp.zeros((16,), jnp.float32)
        if npair > 0:
            prefetch(0, 0)
            prefetch(1, 1)

            def pair_body(cc, acc):
                c = 2 * cc
                for bufb in range(2):
                    for cp in gathers(bufb):
                        cp.wait()
                    acc = compute(bufb, acc)

                    @pl.when(c + 2 + bufb < 2 * npair)
                    def _(c=c, bufb=bufb):
                        prefetch(c + 2 + bufb, bufb)
                return acc

            acc = lax.fori_loop(0, npair, pair_body, acc)

        if tail_groups:
            tb = base + npair * 2 * SCE
            tlen = sum(tail_groups)
            pltpu.sync_copy(src_hbm.at[pl.ds(tb, tlen)],
                            idx_s.at[0, pl.ds(0, tlen)])
            pltpu.sync_copy(dst_hbm.at[pl.ds(tb, tlen)],
                            idx_d.at[0, pl.ds(0, tlen)])
            cps = []
            off = 0
            for j, m in enumerate(tail_groups):
                isl = idx_s.at[0, pl.ds(off, m)]
                dsl = idx_d.at[0, pl.ds(off, m)]
                cps.append(pltpu.make_async_copy(
                    t_hbm.at[isl], rows_s.at[j, pl.ds(0, m)], sem_a))
                cps.append(pltpu.make_async_copy(
                    t2_hbm.at[dsl], rows_d.at[j, pl.ds(0, m)], sem_a))
                off += m
            for cp in cps:
                cp.start()
            for cp in cps:
                cp.wait()
            gsum = jnp.zeros((16,), jnp.float32)
            for j, m in enumerate(tail_groups):
                gsum = compute_group(j, m, gsum)
            acc = acc + gsum

        accv[...] = acc
        pltpu.sync_copy(accv, out_hbm.at[wid])

    return sc_phy


def kernel(pred, target, prev_target, k, x, dt, edge_index):
    b, n = pred.shape[0], pred.shape[1]
    e = edge_index.shape[1]
    b2 = 2 * b

    denom = 2.0 * k * (1.0 - x) + dt
    c0 = (dt - 2.0 * k * x) / denom
    c1 = (dt + 2.0 * k * x) / denom
    c2 = (2.0 * k * (1.0 - x) - dt) / denom
    cvec = jnp.stack([c0, c1, c2]).astype(jnp.float32)

    t_tab, t2_tab, dsum = _make_tc_tables(b2, n)(
        cvec, jnp.reshape(pred, (b, n)), jnp.reshape(target, (b, n)),
        prev_target)

    epad = ((e + NW * 8 - 1) // (NW * 8)) * (NW * 8)
    pad_e = epad - e
    edges = edge_index
    if pad_e:
        edges = jnp.pad(edge_index, ((0, 0), (0, pad_e)))

    acc = _make_sc_phy(epad // NW)(t_tab, t2_tab, edges)

    lane_sums = jnp.sum(acc, axis=0)
    per_batch = lane_sums[:b]
    if pad_e:
        r0 = t2_tab[0, :b] - t_tab[0, :b]
        per_batch = per_batch - jnp.float32(pad_e) * r0 * r0
    phy_loss = jnp.mean(per_batch / jnp.float32(e))
    data_loss = dsum[0, 0] / jnp.float32(b * n)
    total = data_loss + phy_loss
    return (total, data_loss, phy_loss)


# speedup vs baseline: 99.6075x; 1.0799x over previous
import functools

import jax
import jax.numpy as jnp
from jax import lax
from jax.experimental import pallas as pl
from jax.experimental.pallas import tpu as pltpu
from jax.experimental.pallas import tpu_sc as plsc

NC = 2
NS = 16
NW = NC * NS
GROUP = 128
GG = 10
SCE = GROUP * GG
BLK = 512
UNROLL = 16


def _tc_tables_body(n, c_ref, p_ref, t_ref, v_ref, tab_ref, dsum_ref):
    i = pl.program_id(0)
    c0 = c_ref[0]
    c1 = c_ref[1]
    c2 = c_ref[2]
    p = p_ref[...]
    t = t_ref[...]
    v = v_ref[...]
    col = i * BLK + jax.lax.broadcasted_iota(jnp.int32, p.shape, 1)
    valid = col < n
    diff = jnp.where(valid, p - t, 0.0)
    part = jnp.sum(diff * diff)

    @pl.when(i == 0)
    def _():
        dsum_ref[0, 0] = 0.0

    dsum_ref[0, 0] += part
    u = c0 * p + c1 * v
    d = p - c2 * v
    ut = u.T
    dt_ = d.T
    tab_ref[...] = jnp.concatenate([ut, dt_], axis=1)


def _make_tc_tables(b2, n):
    grid = (n + BLK - 1) // BLK
    rows = grid * BLK
    return pl.pallas_call(
        functools.partial(_tc_tables_body, n),
        grid=(grid,),
        in_specs=[
            pl.BlockSpec(memory_space=pltpu.SMEM),
            pl.BlockSpec((b2 // 2, BLK), lambda i: (0, i)),
            pl.BlockSpec((b2 // 2, BLK), lambda i: (0, i)),
            pl.BlockSpec((b2 // 2, BLK), lambda i: (0, i)),
        ],
        out_specs=[
            pl.BlockSpec((BLK, b2), lambda i: (i, 0)),
            pl.BlockSpec((1, 1), lambda i: (0, 0), memory_space=pltpu.SMEM),
        ],
        out_shape=[
            jax.ShapeDtypeStruct((rows, b2), jnp.float32),
            jax.ShapeDtypeStruct((1, 1), jnp.float32),
        ],
    )


def _make_sc_phy(epw):
    npair = epw // (2 * SCE)
    tail = epw - npair * 2 * SCE
    tail_groups = [GROUP] * (tail // GROUP)
    if tail % GROUP:
        tail_groups.append(tail % GROUP)
    assert len(tail_groups) <= 2 * GG
    mesh = plsc.VectorSubcoreMesh(core_axis_name="c", subcore_axis_name="s")

    @functools.partial(
        pl.kernel,
        mesh=mesh,
        compiler_params=pltpu.CompilerParams(use_tc_tiling_on_sc=False),
        out_type=jax.ShapeDtypeStruct((NW, 16), jnp.float32),
        scratch_types=[
            pltpu.VMEM((2, SCE), jnp.int32),
            pltpu.VMEM((2, SCE), jnp.int32),
            pltpu.VMEM((2 * GG, GROUP, 16), jnp.float32),
            pltpu.VMEM((2 * GG, GROUP, 16), jnp.float32),
            pltpu.VMEM((16,), jnp.float32),
            pltpu.SemaphoreType.DMA,
            pltpu.SemaphoreType.DMA,
        ],
    )
    def sc_phy(t_hbm, edge_hbm, out_hbm,
               idx_s, idx_d, rows_s, rows_d, accv, sem_a, sem_b):
        wid = lax.axis_index("s") * NC + lax.axis_index("c")
        base = wid * epw
        sems = (sem_a, sem_b)
        src_hbm = edge_hbm.at[0]
        dst_hbm = edge_hbm.at[1]
        rot8 = lax.iota(jnp.int32, 16) ^ 8

        def gathers(bufb):
            cps = []
            for j in range(GG):
                isl = idx_s.at[bufb, pl.ds(j * GROUP, GROUP)]
                dsl = idx_d.at[bufb, pl.ds(j * GROUP, GROUP)]
                cps.append(pltpu.make_async_copy(
                    t_hbm.at[isl], rows_s.at[bufb * GG + j], sems[bufb]))
                cps.append(pltpu.make_async_copy(
                    t_hbm.at[dsl], rows_d.at[bufb * GG + j], sems[bufb]))
            return cps

        def prefetch(c, bufb):
            gb = base + c * SCE
            pltpu.sync_copy(src_hbm.at[pl.ds(gb, SCE)], idx_s.at[bufb])
            pltpu.sync_copy(dst_hbm.at[pl.ds(gb, SCE)], idx_d.at[bufb])
            for cp in gathers(bufb):
                cp.start()

        def edge_sq(slot, i):
            x = rows_s[slot, i, :][rot8]
            r = rows_d[slot, i, :] - x
            return r * r

        def compute_group(slot, m, gsum):
            def ibody(iv, g, slot=slot):
                for u_ in range(UNROLL):
                    g = g + edge_sq(slot, iv * UNROLL + u_)
                return g

            gsum = lax.fori_loop(0, m // UNROLL, ibody, gsum)
            for i in range(m - (m % UNROLL), m):
                gsum = gsum + edge_sq(slot, i)
            return gsum

        def compute(bufb, acc):
            gsum = jnp.zeros((16,), jnp.float32)
            for j in range(GG):
                gsum = compute_group(bufb * GG + j, GROUP, gsum)
            return acc + gsum

        acc = jnp.zeros((16,), jnp.float32)
        if npair > 0:
            prefetch(0, 0)
            prefetch(1, 1)

            def pair_body(cc, acc):
                c = 2 * cc
                for bufb in range(2):
                    for cp in gathers(bufb):
                        cp.wait()
                    acc = compute(bufb, acc)

                    @pl.when(c + 2 + bufb < 2 * npair)
                    def _(c=c, bufb=bufb):
                        prefetch(c + 2 + bufb, bufb)
                return acc

            acc = lax.fori_loop(0, npair, pair_body, acc)

        if tail_groups:
            tb = base + npair * 2 * SCE
            tlen = sum(tail_groups)
            pltpu.sync_copy(src_hbm.at[pl.ds(tb, tlen)],
                            idx_s.at[0, pl.ds(0, tlen)])
            pltpu.sync_copy(dst_hbm.at[pl.ds(tb, tlen)],
                            idx_d.at[0, pl.ds(0, tlen)])
            cps = []
            off = 0
            for j, m in enumerate(tail_groups):
                isl = idx_s.at[0, pl.ds(off, m)]
                dsl = idx_d.at[0, pl.ds(off, m)]
                cps.append(pltpu.make_async_copy(
                    t_hbm.at[isl], rows_s.at[j, pl.ds(0, m)], sem_a))
                cps.append(pltpu.make_async_copy(
                    t_hbm.at[dsl], rows_d.at[j, pl.ds(0, m)], sem_a))
                off += m
            for cp in cps:
                cp.start()
            for cp in cps:
                cp.wait()
            gsum = jnp.zeros((16,), jnp.float32)
            for j, m in enumerate(tail_groups):
                gsum = compute_group(j, m, gsum)
            acc = acc + gsum

        accv[...] = acc
        pltpu.sync_copy(accv, out_hbm.at[wid])

    return sc_phy


def kernel(pred, target, prev_target, k, x, dt, edge_index):
    b, n = pred.shape[0], pred.shape[1]
    e = edge_index.shape[1]
    b2 = 2 * b

    denom = 2.0 * k * (1.0 - x) + dt
    c0 = (dt - 2.0 * k * x) / denom
    c1 = (dt + 2.0 * k * x) / denom
    c2 = (2.0 * k * (1.0 - x) - dt) / denom
    cvec = jnp.stack([c0, c1, c2]).astype(jnp.float32)

    t_tab, dsum = _make_tc_tables(b2, n)(
        cvec, jnp.reshape(pred, (b, n)), jnp.reshape(target, (b, n)),
        prev_target)

    epad = ((e + NW * 8 - 1) // (NW * 8)) * (NW * 8)
    pad_e = epad - e
    edges = edge_index
    if pad_e:
        edges = jnp.pad(edge_index, ((0, 0), (0, pad_e)))

    acc = _make_sc_phy(epad // NW)(t_tab, edges)

    lane_sums = jnp.sum(acc, axis=0)
    per_batch = lane_sums[b:b2]
    if pad_e:
        r0 = t_tab[0, b:b2] - t_tab[0, :b]
        per_batch = per_batch - jnp.float32(pad_e) * r0 * r0
    phy_loss = jnp.mean(per_batch / jnp.float32(e))
    data_loss = dsum[0, 0] / jnp.float32(b * n)
    total = data_loss + phy_loss
    return (total, data_loss, phy_loss)


# speedup vs baseline: 99.8033x; 1.0020x over previous
import functools

import jax
import jax.numpy as jnp
from jax import lax
from jax.experimental import pallas as pl
from jax.experimental.pallas import tpu as pltpu
from jax.experimental.pallas import tpu_sc as plsc

NC = 2
NS = 16
NW = NC * NS
GROUP = 128
GG = 10
SCE = GROUP * GG
BLK = 512
UNROLL = 16


def _tc_tables_body(n, c_ref, p_ref, t_ref, v_ref, tab_ref, dsum_ref):
    i = pl.program_id(0)
    c0 = c_ref[0]
    c1 = c_ref[1]
    c2 = c_ref[2]
    p = p_ref[...]
    t = t_ref[...]
    v = v_ref[...]
    col = i * BLK + jax.lax.broadcasted_iota(jnp.int32, p.shape, 1)
    valid = col < n
    diff = jnp.where(valid, p - t, 0.0)
    part = jnp.sum(diff * diff)

    @pl.when(i == 0)
    def _():
        dsum_ref[0, 0] = 0.0

    dsum_ref[0, 0] += part
    u = c0 * p + c1 * v
    d = p - c2 * v
    tab_ref[...] = jnp.concatenate([u, d], axis=0)


def _make_tc_tables(b2, n):
    grid = (n + BLK - 1) // BLK
    cols = grid * BLK
    return pl.pallas_call(
        functools.partial(_tc_tables_body, n),
        grid=(grid,),
        in_specs=[
            pl.BlockSpec(memory_space=pltpu.SMEM),
            pl.BlockSpec((b2 // 2, BLK), lambda i: (0, i)),
            pl.BlockSpec((b2 // 2, BLK), lambda i: (0, i)),
            pl.BlockSpec((b2 // 2, BLK), lambda i: (0, i)),
        ],
        out_specs=[
            pl.BlockSpec((b2, BLK), lambda i: (0, i)),
            pl.BlockSpec((1, 1), lambda i: (0, 0), memory_space=pltpu.SMEM),
        ],
        out_shape=[
            jax.ShapeDtypeStruct((b2, cols), jnp.float32),
            jax.ShapeDtypeStruct((1, 1), jnp.float32),
        ],
    )


def _make_sc_phy(epw):
    npair = epw // (2 * SCE)
    tail = epw - npair * 2 * SCE
    tail_groups = [GROUP] * (tail // GROUP)
    if tail % GROUP:
        tail_groups.append(tail % GROUP)
    assert len(tail_groups) <= 2 * GG
    mesh = plsc.VectorSubcoreMesh(core_axis_name="c", subcore_axis_name="s")

    @functools.partial(
        pl.kernel,
        mesh=mesh,
        compiler_params=pltpu.CompilerParams(use_tc_tiling_on_sc=False),
        out_type=jax.ShapeDtypeStruct((NW, 16), jnp.float32),
        scratch_types=[
            pltpu.VMEM((2, SCE), jnp.int32),
            pltpu.VMEM((2, SCE), jnp.int32),
            pltpu.VMEM((2 * GG, GROUP, 16), jnp.float32),
            pltpu.VMEM((2 * GG, GROUP, 16), jnp.float32),
            pltpu.VMEM((16,), jnp.float32),
            pltpu.SemaphoreType.DMA,
            pltpu.SemaphoreType.DMA,
        ],
    )
    def sc_phy(t_hbm, edge_hbm, out_hbm,
               idx_s, idx_d, rows_s, rows_d, accv, sem_a, sem_b):
        wid = lax.axis_index("s") * NC + lax.axis_index("c")
        base = wid * epw
        sems = (sem_a, sem_b)
        src_hbm = edge_hbm.at[0]
        dst_hbm = edge_hbm.at[1]
        rot8 = lax.iota(jnp.int32, 16) ^ 8

        def gathers(bufb):
            cps = []
            for j in range(GG):
                isl = idx_s.at[bufb, pl.ds(j * GROUP, GROUP)]
                dsl = idx_d.at[bufb, pl.ds(j * GROUP, GROUP)]
                cps.append(pltpu.make_async_copy(
                    t_hbm.at[isl], rows_s.at[bufb * GG + j], sems[bufb]))
                cps.append(pltpu.make_async_copy(
                    t_hbm.at[dsl], rows_d.at[bufb * GG + j], sems[bufb]))
            return cps

        def prefetch(c, bufb):
            gb = base + c * SCE
            pltpu.sync_copy(src_hbm.at[pl.ds(gb, SCE)], idx_s.at[bufb])
            pltpu.sync_copy(dst_hbm.at[pl.ds(gb, SCE)], idx_d.at[bufb])
            for cp in gathers(bufb):
                cp.start()

        def edge_sq(slot, i):
            x = rows_s[slot, i, :][rot8]
            r = rows_d[slot, i, :] - x
            return r * r

        def compute_group(slot, m, gsum):
            def ibody(iv, g, slot=slot):
                for u_ in range(UNROLL):
                    g = g + edge_sq(slot, iv * UNROLL + u_)
                return g

            gsum = lax.fori_loop(0, m // UNROLL, ibody, gsum)
            for i in range(m - (m % UNROLL), m):
                gsum = gsum + edge_sq(slot, i)
            return gsum

        def compute(bufb, acc):
            gsum = jnp.zeros((16,), jnp.float32)
            for j in range(GG):
                gsum = compute_group(bufb * GG + j, GROUP, gsum)
            return acc + gsum

        acc = jnp.zeros((16,), jnp.float32)
        if npair > 0:
            prefetch(0, 0)
            prefetch(1, 1)

            def pair_body(cc, acc):
                c = 2 * cc
                for bufb in range(2):
                    for cp in gathers(bufb):
                        cp.wait()
                    acc = compute(bufb, acc)

                    @pl.when(c + 2 + bufb < 2 * npair)
                    def _(c=c, bufb=bufb):
                        prefetch(c + 2 + bufb, bufb)
                return acc

            acc = lax.fori_loop(0, npair, pair_body, acc)

        if tail_groups:
            tb = base + npair * 2 * SCE
            tlen = sum(tail_groups)
            pltpu.sync_copy(src_hbm.at[pl.ds(tb, tlen)],
                            idx_s.at[0, pl.ds(0, tlen)])
            pltpu.sync_copy(dst_hbm.at[pl.ds(tb, tlen)],
                            idx_d.at[0, pl.ds(0, tlen)])
            cps = []
            off = 0
            for j, m in enumerate(tail_groups):
                isl = idx_s.at[0, pl.ds(off, m)]
                dsl = idx_d.at[0, pl.ds(off, m)]
                cps.append(pltpu.make_async_copy(
                    t_hbm.at[isl], rows_s.at[j, pl.ds(0, m)], sem_a))
                cps.append(pltpu.make_async_copy(
                    t_hbm.at[dsl], rows_d.at[j, pl.ds(0, m)], sem_a))
                off += m
            for cp in cps:
                cp.start()
            for cp in cps:
                cp.wait()
            gsum = jnp.zeros((16,), jnp.float32)
            for j, m in enumerate(tail_groups):
                gsum = compute_group(j, m, gsum)
            acc = acc + gsum

        accv[...] = acc
        pltpu.sync_copy(accv, out_hbm.at[wid])

    return sc_phy


def kernel(pred, target, prev_target, k, x, dt, edge_index):
    b, n = pred.shape[0], pred.shape[1]
    e = edge_index.shape[1]
    b2 = 2 * b

    denom = 2.0 * k * (1.0 - x) + dt
    c0 = (dt - 2.0 * k * x) / denom
    c1 = (dt + 2.0 * k * x) / denom
    c2 = (2.0 * k * (1.0 - x) - dt) / denom
    cvec = jnp.stack([c0, c1, c2]).astype(jnp.float32)

    ud, dsum = _make_tc_tables(b2, n)(
        cvec, jnp.reshape(pred, (b, n)), jnp.reshape(target, (b, n)),
        prev_target)
    t_tab = ud.T

    epad = ((e + NW * 8 - 1) // (NW * 8)) * (NW * 8)
    pad_e = epad - e
    edges = edge_index
    if pad_e:
        edges = jnp.pad(edge_index, ((0, 0), (0, pad_e)))

    acc = _make_sc_phy(epad // NW)(t_tab, edges)

    lane_sums = jnp.sum(acc, axis=0)
    per_batch = lane_sums[b:b2]
    if pad_e:
        r0 = t_tab[0, b:b2] - t_tab[0, :b]
        per_batch = per_batch - jnp.float32(pad_e) * r0 * r0
    phy_loss = jnp.mean(per_batch / jnp.float32(e))
    data_loss = dsum[0, 0] / jnp.float32(b * n)
    total = data_loss + phy_loss
    return (total, data_loss, phy_loss)


# speedup vs baseline: 120.6186x; 1.2086x over previous
import functools

import jax
import jax.numpy as jnp
from jax import lax
from jax.experimental import pallas as pl
from jax.experimental.pallas import tpu as pltpu
from jax.experimental.pallas import tpu_sc as plsc

NC = 2
NS = 16
NW = NC * NS
GROUP = 128
GG = 10
SCE = GROUP * GG
BLK = 4096
UNROLL = 16


def _tc_tables_body(n, c_ref, p_ref, t_ref, v_ref, tab_ref, dsum_ref):
    i = pl.program_id(0)
    c0 = c_ref[0]
    c1 = c_ref[1]
    c2 = c_ref[2]
    p = p_ref[...]
    t = t_ref[...]
    v = v_ref[...]
    col = i * BLK + jax.lax.broadcasted_iota(jnp.int32, p.shape, 1)
    valid = col < n
    diff = jnp.where(valid, p - t, 0.0)
    part = jnp.sum(diff * diff)

    @pl.when(i == 0)
    def _():
        dsum_ref[0, 0] = 0.0

    dsum_ref[0, 0] += part
    u = c0 * p + c1 * v
    d = p - c2 * v
    tab_ref[...] = jnp.concatenate([u, d], axis=0)


def _make_tc_tables(b2, n):
    grid = (n + BLK - 1) // BLK
    cols = grid * BLK
    return pl.pallas_call(
        functools.partial(_tc_tables_body, n),
        grid=(grid,),
        in_specs=[
            pl.BlockSpec(memory_space=pltpu.SMEM),
            pl.BlockSpec((b2 // 2, BLK), lambda i: (0, i)),
            pl.BlockSpec((b2 // 2, BLK), lambda i: (0, i)),
            pl.BlockSpec((b2 // 2, BLK), lambda i: (0, i)),
        ],
        out_specs=[
            pl.BlockSpec((b2, BLK), lambda i: (0, i)),
            pl.BlockSpec((1, 1), lambda i: (0, 0), memory_space=pltpu.SMEM),
        ],
        out_shape=[
            jax.ShapeDtypeStruct((b2, cols), jnp.float32),
            jax.ShapeDtypeStruct((1, 1), jnp.float32),
        ],
    )


def _make_sc_phy(epw):
    npair = epw // (2 * SCE)
    tail = epw - npair * 2 * SCE
    tail_groups = [GROUP] * (tail // GROUP)
    if tail % GROUP:
        tail_groups.append(tail % GROUP)
    assert len(tail_groups) <= 2 * GG
    mesh = plsc.VectorSubcoreMesh(core_axis_name="c", subcore_axis_name="s")

    @functools.partial(
        pl.kernel,
        mesh=mesh,
        compiler_params=pltpu.CompilerParams(use_tc_tiling_on_sc=False),
        out_type=jax.ShapeDtypeStruct((NW, 16), jnp.float32),
        scratch_types=[
            pltpu.VMEM((2, SCE), jnp.int32),
            pltpu.VMEM((2, SCE), jnp.int32),
            pltpu.VMEM((2 * GG, GROUP, 16), jnp.float32),
            pltpu.VMEM((2 * GG, GROUP, 16), jnp.float32),
            pltpu.VMEM((16,), jnp.float32),
            pltpu.SemaphoreType.DMA,
            pltpu.SemaphoreType.DMA,
        ],
    )
    def sc_phy(t_hbm, edge_hbm, out_hbm,
               idx_s, idx_d, rows_s, rows_d, accv, sem_a, sem_b):
        wid = lax.axis_index("s") * NC + lax.axis_index("c")
        base = wid * epw
        sems = (sem_a, sem_b)
        src_hbm = edge_hbm.at[0]
        dst_hbm = edge_hbm.at[1]
        rot8 = lax.iota(jnp.int32, 16) ^ 8

        def gathers(bufb):
            cps = []
            for j in range(GG):
                isl = idx_s.at[bufb, pl.ds(j * GROUP, GROUP)]
                dsl = idx_d.at[bufb, pl.ds(j * GROUP, GROUP)]
                cps.append(pltpu.make_async_copy(
                    t_hbm.at[isl], rows_s.at[bufb * GG + j], sems[bufb]))
                cps.append(pltpu.make_async_copy(
                    t_hbm.at[dsl], rows_d.at[bufb * GG + j], sems[bufb]))
            return cps

        def prefetch(c, bufb):
            gb = base + c * SCE
            pltpu.sync_copy(src_hbm.at[pl.ds(gb, SCE)], idx_s.at[bufb])
            pltpu.sync_copy(dst_hbm.at[pl.ds(gb, SCE)], idx_d.at[bufb])
            for cp in gathers(bufb):
                cp.start()

        def edge_sq(slot, i):
            x = rows_s[slot, i, :][rot8]
            r = rows_d[slot, i, :] - x
            return r * r

        def compute_group(slot, m, gsum):
            def ibody(iv, g, slot=slot):
                for u_ in range(UNROLL):
                    g = g + edge_sq(slot, iv * UNROLL + u_)
                return g

            gsum = lax.fori_loop(0, m // UNROLL, ibody, gsum)
            for i in range(m - (m % UNROLL), m):
                gsum = gsum + edge_sq(slot, i)
            return gsum

        def compute(bufb, acc):
            gsum = jnp.zeros((16,), jnp.float32)
            for j in range(GG):
                gsum = compute_group(bufb * GG + j, GROUP, gsum)
            return acc + gsum

        acc = jnp.zeros((16,), jnp.float32)
        if npair > 0:
            prefetch(0, 0)
            prefetch(1, 1)

            def pair_body(cc, acc):
                c = 2 * cc
                for bufb in range(2):
                    for cp in gathers(bufb):
                        cp.wait()
                    acc = compute(bufb, acc)

                    @pl.when(c + 2 + bufb < 2 * npair)
                    def _(c=c, bufb=bufb):
                        prefetch(c + 2 + bufb, bufb)
                return acc

            acc = lax.fori_loop(0, npair, pair_body, acc)

        if tail_groups:
            tb = base + npair * 2 * SCE
            tlen = sum(tail_groups)
            pltpu.sync_copy(src_hbm.at[pl.ds(tb, tlen)],
                            idx_s.at[0, pl.ds(0, tlen)])
            pltpu.sync_copy(dst_hbm.at[pl.ds(tb, tlen)],
                            idx_d.at[0, pl.ds(0, tlen)])
            cps = []
            off = 0
            for j, m in enumerate(tail_groups):
                isl = idx_s.at[0, pl.ds(off, m)]
                dsl = idx_d.at[0, pl.ds(off, m)]
                cps.append(pltpu.make_async_copy(
                    t_hbm.at[isl], rows_s.at[j, pl.ds(0, m)], sem_a))
                cps.append(pltpu.make_async_copy(
                    t_hbm.at[dsl], rows_d.at[j, pl.ds(0, m)], sem_a))
                off += m
            for cp in cps:
                cp.start()
            for cp in cps:
                cp.wait()
            gsum = jnp.zeros((16,), jnp.float32)
            for j, m in enumerate(tail_groups):
                gsum = compute_group(j, m, gsum)
            acc = acc + gsum

        accv[...] = acc
        pltpu.sync_copy(accv, out_hbm.at[wid])

    return sc_phy


def kernel(pred, target, prev_target, k, x, dt, edge_index):
    b, n = pred.shape[0], pred.shape[1]
    e = edge_index.shape[1]
    b2 = 2 * b

    denom = 2.0 * k * (1.0 - x) + dt
    c0 = (dt - 2.0 * k * x) / denom
    c1 = (dt + 2.0 * k * x) / denom
    c2 = (2.0 * k * (1.0 - x) - dt) / denom
    cvec = jnp.stack([c0, c1, c2]).astype(jnp.float32)

    ud, dsum = _make_tc_tables(b2, n)(
        cvec, jnp.reshape(pred, (b, n)), jnp.reshape(target, (b, n)),
        prev_target)
    t_tab = ud.T

    epad = ((e + NW * 8 - 1) // (NW * 8)) * (NW * 8)
    pad_e = epad - e
    edges = edge_index
    if pad_e:
        edges = jnp.pad(edge_index, ((0, 0), (0, pad_e)))

    acc = _make_sc_phy(epad // NW)(t_tab, edges)

    lane_sums = jnp.sum(acc, axis=0)
    per_batch = lane_sums[b:b2]
    if pad_e:
        r0 = t_tab[0, b:b2] - t_tab[0, :b]
        per_batch = per_batch - jnp.float32(pad_e) * r0 * r0
    phy_loss = jnp.mean(per_batch / jnp.float32(e))
    data_loss = dsum[0, 0] / jnp.float32(b * n)
    total = data_loss + phy_loss
    return (total, data_loss, phy_loss)


# speedup vs baseline: 121.2489x; 1.0052x over previous
import functools

import jax
import jax.numpy as jnp
from jax import lax
from jax.experimental import pallas as pl
from jax.experimental.pallas import tpu as pltpu
from jax.experimental.pallas import tpu_sc as plsc

NC = 2
NS = 16
NW = NC * NS
GROUP = 128
GG = 14
SCE = GROUP * GG
BLK = 4096
UNROLL = 16


def _tc_tables_body(n, c_ref, p_ref, t_ref, v_ref, tab_ref, dsum_ref):
    i = pl.program_id(0)
    c0 = c_ref[0]
    c1 = c_ref[1]
    c2 = c_ref[2]
    p = p_ref[...]
    t = t_ref[...]
    v = v_ref[...]
    col = i * BLK + jax.lax.broadcasted_iota(jnp.int32, p.shape, 1)
    valid = col < n
    diff = jnp.where(valid, p - t, 0.0)
    part = jnp.sum(diff * diff)

    @pl.when(i == 0)
    def _():
        dsum_ref[0, 0] = 0.0

    dsum_ref[0, 0] += part
    u = c0 * p + c1 * v
    d = p - c2 * v
    tab_ref[...] = jnp.concatenate([u, d], axis=0)


def _make_tc_tables(b2, n):
    grid = (n + BLK - 1) // BLK
    cols = grid * BLK
    return pl.pallas_call(
        functools.partial(_tc_tables_body, n),
        grid=(grid,),
        in_specs=[
            pl.BlockSpec(memory_space=pltpu.SMEM),
            pl.BlockSpec((b2 // 2, BLK), lambda i: (0, i)),
            pl.BlockSpec((b2 // 2, BLK), lambda i: (0, i)),
            pl.BlockSpec((b2 // 2, BLK), lambda i: (0, i)),
        ],
        out_specs=[
            pl.BlockSpec((b2, BLK), lambda i: (0, i)),
            pl.BlockSpec((1, 1), lambda i: (0, 0), memory_space=pltpu.SMEM),
        ],
        out_shape=[
            jax.ShapeDtypeStruct((b2, cols), jnp.float32),
            jax.ShapeDtypeStruct((1, 1), jnp.float32),
        ],
    )


def _make_sc_phy(epw):
    npair = epw // (2 * SCE)
    tail = epw - npair * 2 * SCE
    tail_groups = [GROUP] * (tail // GROUP)
    if tail % GROUP:
        tail_groups.append(tail % GROUP)
    assert len(tail_groups) <= 2 * GG
    mesh = plsc.VectorSubcoreMesh(core_axis_name="c", subcore_axis_name="s")

    @functools.partial(
        pl.kernel,
        mesh=mesh,
        compiler_params=pltpu.CompilerParams(use_tc_tiling_on_sc=False),
        out_type=jax.ShapeDtypeStruct((NW, 16), jnp.float32),
        scratch_types=[
            pltpu.VMEM((2, SCE), jnp.int32),
            pltpu.VMEM((2, SCE), jnp.int32),
            pltpu.VMEM((2 * GG, GROUP, 16), jnp.float32),
            pltpu.VMEM((2 * GG, GROUP, 16), jnp.float32),
            pltpu.VMEM((16,), jnp.float32),
            pltpu.SemaphoreType.DMA,
            pltpu.SemaphoreType.DMA,
        ],
    )
    def sc_phy(t_hbm, edge_hbm, out_hbm,
               idx_s, idx_d, rows_s, rows_d, accv, sem_a, sem_b):
        wid = lax.axis_index("s") * NC + lax.axis_index("c")
        base = wid * epw
        sems = (sem_a, sem_b)
        src_hbm = edge_hbm.at[0]
        dst_hbm = edge_hbm.at[1]
        rot8 = lax.iota(jnp.int32, 16) ^ 8

        def gathers(bufb):
            cps = []
            for j in range(GG):
                isl = idx_s.at[bufb, pl.ds(j * GROUP, GROUP)]
                dsl = idx_d.at[bufb, pl.ds(j * GROUP, GROUP)]
                cps.append(pltpu.make_async_copy(
                    t_hbm.at[isl], rows_s.at[bufb * GG + j], sems[bufb]))
                cps.append(pltpu.make_async_copy(
                    t_hbm.at[dsl], rows_d.at[bufb * GG + j], sems[bufb]))
            return cps

        def prefetch(c, bufb):
            gb = base + c * SCE
            pltpu.sync_copy(src_hbm.at[pl.ds(gb, SCE)], idx_s.at[bufb])
            pltpu.sync_copy(dst_hbm.at[pl.ds(gb, SCE)], idx_d.at[bufb])
            for cp in gathers(bufb):
                cp.start()

        def edge_sq(slot, i):
            x = rows_s[slot, i, :][rot8]
            r = rows_d[slot, i, :] - x
            return r * r

        def compute_group(slot, m, gsum):
            def ibody(iv, g, slot=slot):
                for u_ in range(UNROLL):
                    g = g + edge_sq(slot, iv * UNROLL + u_)
                return g

            gsum = lax.fori_loop(0, m // UNROLL, ibody, gsum)
            for i in range(m - (m % UNROLL), m):
                gsum = gsum + edge_sq(slot, i)
            return gsum

        def compute(bufb, acc):
            gsum = jnp.zeros((16,), jnp.float32)
            for j in range(GG):
                gsum = compute_group(bufb * GG + j, GROUP, gsum)
            return acc + gsum

        acc = jnp.zeros((16,), jnp.float32)
        if npair > 0:
            prefetch(0, 0)
            prefetch(1, 1)

            def pair_body(cc, acc):
                c = 2 * cc
                for bufb in range(2):
                    for cp in gathers(bufb):
                        cp.wait()
                    acc = compute(bufb, acc)

                    @pl.when(c + 2 + bufb < 2 * npair)
                    def _(c=c, bufb=bufb):
                        prefetch(c + 2 + bufb, bufb)
                return acc

            acc = lax.fori_loop(0, npair, pair_body, acc)

        if tail_groups:
            tb = base + npair * 2 * SCE
            tlen = sum(tail_groups)
            tlen0 = min(tlen, SCE)
            pltpu.sync_copy(src_hbm.at[pl.ds(tb, tlen0)],
                            idx_s.at[0, pl.ds(0, tlen0)])
            pltpu.sync_copy(dst_hbm.at[pl.ds(tb, tlen0)],
                            idx_d.at[0, pl.ds(0, tlen0)])
            if tlen > tlen0:
                pltpu.sync_copy(src_hbm.at[pl.ds(tb + tlen0, tlen - tlen0)],
                                idx_s.at[1, pl.ds(0, tlen - tlen0)])
                pltpu.sync_copy(dst_hbm.at[pl.ds(tb + tlen0, tlen - tlen0)],
                                idx_d.at[1, pl.ds(0, tlen - tlen0)])
            cps = []
            off = 0
            for j, m in enumerate(tail_groups):
                rowb, roff = off // SCE, off % SCE
                isl = idx_s.at[rowb, pl.ds(roff, m)]
                dsl = idx_d.at[rowb, pl.ds(roff, m)]
                cps.append(pltpu.make_async_copy(
                    t_hbm.at[isl], rows_s.at[j, pl.ds(0, m)], sem_a))
                cps.append(pltpu.make_async_copy(
                    t_hbm.at[dsl], rows_d.at[j, pl.ds(0, m)], sem_a))
                off += m
            for cp in cps:
                cp.start()
            for cp in cps:
                cp.wait()
            gsum = jnp.zeros((16,), jnp.float32)
            for j, m in enumerate(tail_groups):
                gsum = compute_group(j, m, gsum)
            acc = acc + gsum

        accv[...] = acc
        pltpu.sync_copy(accv, out_hbm.at[wid])

    return sc_phy


def kernel(pred, target, prev_target, k, x, dt, edge_index):
    b, n = pred.shape[0], pred.shape[1]
    e = edge_index.shape[1]
    b2 = 2 * b

    denom = 2.0 * k * (1.0 - x) + dt
    c0 = (dt - 2.0 * k * x) / denom
    c1 = (dt + 2.0 * k * x) / denom
    c2 = (2.0 * k * (1.0 - x) - dt) / denom
    cvec = jnp.stack([c0, c1, c2]).astype(jnp.float32)

    ud, dsum = _make_tc_tables(b2, n)(
        cvec, jnp.reshape(pred, (b, n)), jnp.reshape(target, (b, n)),
        prev_target)
    t_tab = ud.T

    epad = ((e + NW * 8 - 1) // (NW * 8)) * (NW * 8)
    pad_e = epad - e
    edges = edge_index
    if pad_e:
        edges = jnp.pad(edge_index, ((0, 0), (0, pad_e)))

    acc = _make_sc_phy(epad // NW)(t_tab, edges)

    lane_sums = jnp.sum(acc, axis=0)
    per_batch = lane_sums[b:b2]
    if pad_e:
        r0 = t_tab[0, b:b2] - t_tab[0, :b]
        per_batch = per_batch - jnp.float32(pad_e) * r0 * r0
    phy_loss = jnp.mean(per_batch / jnp.float32(e))
    data_loss = dsum[0, 0] / jnp.float32(b * n)
    total = data_loss + phy_loss
    return (total, data_loss, phy_loss)


# speedup vs baseline: 125.1021x; 1.0318x over previous
import functools

import jax
import jax.numpy as jnp
from jax import lax
from jax.experimental import pallas as pl
from jax.experimental.pallas import tpu as pltpu
from jax.experimental.pallas import tpu_sc as plsc

NC = 2
NS = 16
NW = NC * NS
GROUP = 128
GG = 14
SCE = GROUP * GG
BLK = 4096
UNROLL = 16


def _tc_tables_body(n, c_ref, p_ref, t_ref, v_ref, tab_ref, dsum_ref):
    i = pl.program_id(0)
    c0 = c_ref[0]
    c1 = c_ref[1]
    c2 = c_ref[2]
    p = p_ref[...]
    t = t_ref[...]
    v = v_ref[...]
    col = i * BLK + jax.lax.broadcasted_iota(jnp.int32, p.shape, 1)
    valid = col < n
    diff = jnp.where(valid, p - t, 0.0)
    part = jnp.sum(diff * diff)

    @pl.when(i == 0)
    def _():
        dsum_ref[0, 0] = 0.0

    dsum_ref[0, 0] += part
    u = c0 * p + c1 * v
    d = p - c2 * v
    val = jnp.concatenate([u, d], axis=0)
    eye = (jax.lax.broadcasted_iota(jnp.int32, (16, 16), 0)
           == jax.lax.broadcasted_iota(jnp.int32, (16, 16), 1)
           ).astype(jnp.float32)
    tab_ref[...] = jax.lax.dot_general(
        val, eye, (((0,), (0,)), ((), ())),
        preferred_element_type=jnp.float32)


def _make_tc_tables(b2, n):
    grid = (n + BLK - 1) // BLK
    cols = grid * BLK
    return pl.pallas_call(
        functools.partial(_tc_tables_body, n),
        grid=(grid,),
        in_specs=[
            pl.BlockSpec(memory_space=pltpu.SMEM),
            pl.BlockSpec((b2 // 2, BLK), lambda i: (0, i)),
            pl.BlockSpec((b2 // 2, BLK), lambda i: (0, i)),
            pl.BlockSpec((b2 // 2, BLK), lambda i: (0, i)),
        ],
        out_specs=[
            pl.BlockSpec((BLK, b2), lambda i: (i, 0)),
            pl.BlockSpec((1, 1), lambda i: (0, 0), memory_space=pltpu.SMEM),
        ],
        out_shape=[
            jax.ShapeDtypeStruct((cols, b2), jnp.float32),
            jax.ShapeDtypeStruct((1, 1), jnp.float32),
        ],
    )


def _make_sc_phy(epw):
    npair = epw // (2 * SCE)
    tail = epw - npair * 2 * SCE
    tail_groups = [GROUP] * (tail // GROUP)
    if tail % GROUP:
        tail_groups.append(tail % GROUP)
    assert len(tail_groups) <= 2 * GG
    mesh = plsc.VectorSubcoreMesh(core_axis_name="c", subcore_axis_name="s")

    @functools.partial(
        pl.kernel,
        mesh=mesh,
        compiler_params=pltpu.CompilerParams(use_tc_tiling_on_sc=False),
        out_type=jax.ShapeDtypeStruct((NW, 16), jnp.float32),
        scratch_types=[
            pltpu.VMEM((2, SCE), jnp.int32),
            pltpu.VMEM((2, SCE), jnp.int32),
            pltpu.VMEM((2 * GG, GROUP, 16), jnp.float32),
            pltpu.VMEM((2 * GG, GROUP, 16), jnp.float32),
            pltpu.VMEM((16,), jnp.float32),
            pltpu.SemaphoreType.DMA,
            pltpu.SemaphoreType.DMA,
        ],
    )
    def sc_phy(t_hbm, edge_hbm, out_hbm,
               idx_s, idx_d, rows_s, rows_d, accv, sem_a, sem_b):
        wid = lax.axis_index("s") * NC + lax.axis_index("c")
        base = wid * epw
        sems = (sem_a, sem_b)
        src_hbm = edge_hbm.at[0]
        dst_hbm = edge_hbm.at[1]
        rot8 = lax.iota(jnp.int32, 16) ^ 8

        def gathers(bufb):
            cps = []
            for j in range(GG):
                isl = idx_s.at[bufb, pl.ds(j * GROUP, GROUP)]
                dsl = idx_d.at[bufb, pl.ds(j * GROUP, GROUP)]
                cps.append(pltpu.make_async_copy(
                    t_hbm.at[isl], rows_s.at[bufb * GG + j], sems[bufb]))
                cps.append(pltpu.make_async_copy(
                    t_hbm.at[dsl], rows_d.at[bufb * GG + j], sems[bufb]))
            return cps

        def prefetch(c, bufb):
            gb = base + c * SCE
            pltpu.sync_copy(src_hbm.at[pl.ds(gb, SCE)], idx_s.at[bufb])
            pltpu.sync_copy(dst_hbm.at[pl.ds(gb, SCE)], idx_d.at[bufb])
            for cp in gathers(bufb):
                cp.start()

        def edge_sq(slot, i):
            x = rows_s[slot, i, :][rot8]
            r = rows_d[slot, i, :] - x
            return r * r

        def compute_group(slot, m, gsum):
            def ibody(iv, g, slot=slot):
                for u_ in range(UNROLL):
                    g = g + edge_sq(slot, iv * UNROLL + u_)
                return g

            gsum = lax.fori_loop(0, m // UNROLL, ibody, gsum)
            for i in range(m - (m % UNROLL), m):
                gsum = gsum + edge_sq(slot, i)
            return gsum

        def compute(bufb, acc):
            gsum = jnp.zeros((16,), jnp.float32)
            for j in range(GG):
                gsum = compute_group(bufb * GG + j, GROUP, gsum)
            return acc + gsum

        acc = jnp.zeros((16,), jnp.float32)
        if npair > 0:
            prefetch(0, 0)
            prefetch(1, 1)

            def pair_body(cc, acc):
                c = 2 * cc
                for bufb in range(2):
                    for cp in gathers(bufb):
                        cp.wait()
                    acc = compute(bufb, acc)

                    @pl.when(c + 2 + bufb < 2 * npair)
                    def _(c=c, bufb=bufb):
                        prefetch(c + 2 + bufb, bufb)
                return acc

            acc = lax.fori_loop(0, npair, pair_body, acc)

        if tail_groups:
            tb = base + npair * 2 * SCE
            tlen = sum(tail_groups)
            tlen0 = min(tlen, SCE)
            pltpu.sync_copy(src_hbm.at[pl.ds(tb, tlen0)],
                            idx_s.at[0, pl.ds(0, tlen0)])
            pltpu.sync_copy(dst_hbm.at[pl.ds(tb, tlen0)],
                            idx_d.at[0, pl.ds(0, tlen0)])
            if tlen > tlen0:
                pltpu.sync_copy(src_hbm.at[pl.ds(tb + tlen0, tlen - tlen0)],
                                idx_s.at[1, pl.ds(0, tlen - tlen0)])
                pltpu.sync_copy(dst_hbm.at[pl.ds(tb + tlen0, tlen - tlen0)],
                                idx_d.at[1, pl.ds(0, tlen - tlen0)])
            cps = []
            off = 0
            for j, m in enumerate(tail_groups):
                rowb, roff = off // SCE, off % SCE
                isl = idx_s.at[rowb, pl.ds(roff, m)]
                dsl = idx_d.at[rowb, pl.ds(roff, m)]
                cps.append(pltpu.make_async_copy(
                    t_hbm.at[isl], rows_s.at[j, pl.ds(0, m)], sem_a))
                cps.append(pltpu.make_async_copy(
                    t_hbm.at[dsl], rows_d.at[j, pl.ds(0, m)], sem_a))
                off += m
            for cp in cps:
                cp.start()
            for cp in cps:
                cp.wait()
            gsum = jnp.zeros((16,), jnp.float32)
            for j, m in enumerate(tail_groups):
                gsum = compute_group(j, m, gsum)
            acc = acc + gsum

        accv[...] = acc
        pltpu.sync_copy(accv, out_hbm.at[wid])

    return sc_phy


def kernel(pred, target, prev_target, k, x, dt, edge_index):
    b, n = pred.shape[0], pred.shape[1]
    e = edge_index.shape[1]
    b2 = 2 * b

    denom = 2.0 * k * (1.0 - x) + dt
    c0 = (dt - 2.0 * k * x) / denom
    c1 = (dt + 2.0 * k * x) / denom
    c2 = (2.0 * k * (1.0 - x) - dt) / denom
    cvec = jnp.stack([c0, c1, c2]).astype(jnp.float32)

    t_tab, dsum = _make_tc_tables(b2, n)(
        cvec, jnp.reshape(pred, (b, n)), jnp.reshape(target, (b, n)),
        prev_target)

    epad = ((e + NW * 8 - 1) // (NW * 8)) * (NW * 8)
    pad_e = epad - e
    edges = edge_index
    if pad_e:
        edges = jnp.pad(edge_index, ((0, 0), (0, pad_e)))

    acc = _make_sc_phy(epad // NW)(t_tab, edges)

    lane_sums = jnp.sum(acc, axis=0)
    per_batch = lane_sums[b:b2]
    if pad_e:
        r0 = t_tab[0, b:b2] - t_tab[0, :b]
        per_batch = per_batch - jnp.float32(pad_e) * r0 * r0
    phy_loss = jnp.mean(per_batch / jnp.float32(e))
    data_loss = dsum[0, 0] / jnp.float32(b * n)
    total = data_loss + phy_loss
    return (total, data_loss, phy_loss)
